# baseline (speedup 1.0000x reference)
import functools

import jax
import jax.numpy as jnp
from jax import lax
from jax.experimental import pallas as pl
from jax.experimental.pallas import tpu as pltpu
from jax.experimental.pallas import tpu_sc as plsc

VOCAB = 32000
HIDDEN = 768
MAX_POS = 64
BATCH = 1024
SEQ = 64

_NC = 2
_NS = 16
_NW = _NC * _NS
_B = BATCH * SEQ
_BPW = _B // _NW
_CH = 64
_NCH = _BPW // _CH
_VECS = HIDDEN // 16


def _make_sc_kernel():
    mesh = plsc.VectorSubcoreMesh(core_axis_name="c", subcore_axis_name="s")

    @functools.partial(
        pl.kernel,
        mesh=mesh,
        out_type=jax.ShapeDtypeStruct((_B, HIDDEN), jnp.float32),
        scratch_types=[
            pltpu.VMEM((_BPW,), jnp.int32),
            pltpu.VMEM((MAX_POS, HIDDEN), jnp.float32),
            pltpu.VMEM((_CH, HIDDEN), jnp.float32),
            pltpu.SemaphoreType.DMA,
        ],
    )
    def embed(ids_hbm, tok_hbm, pos_hbm, out_hbm, idx_v, pos_v, rows_v, sem):
        wid = lax.axis_index("s") * _NC + lax.axis_index("c")
        base = wid * _BPW
        pltpu.sync_copy(ids_hbm.at[pl.ds(base, _BPW)], idx_v)
        pltpu.sync_copy(pos_hbm, pos_v)

        def chunk_body(c, carry):
            pltpu.async_copy(
                tok_hbm.at[idx_v.at[pl.ds(c * _CH, _CH)]], rows_v, sem
            ).wait()

            def row_body(r, rcarry):
                for h in range(_VECS):
                    sl = pl.ds(h * 16, 16)
                    rows_v[r, sl] = rows_v[r, sl] + pos_v[r, sl]
                return rcarry

            lax.fori_loop(0, _CH, row_body, 0)
            pltpu.sync_copy(rows_v, out_hbm.at[pl.ds(base + c * _CH, _CH)])
            return carry

        lax.fori_loop(0, _NCH, chunk_body, 0)

    return embed


_sc_embed = _make_sc_kernel()


def kernel(input_ids, token_embedding, position_embedding):
    ids = input_ids.reshape(_B).astype(jnp.int32)
    out = _sc_embed(ids, token_embedding, position_embedding)
    return out.reshape(BATCH, SEQ, HIDDEN)
